# baseline (speedup 1.0000x reference)
import functools

import jax
import jax.numpy as jnp
from jax.experimental import pallas as pl
from jax.experimental.pallas import tpu as pltpu

_CB = 1024
_D = 32
_TB = 512
_N = 16 * 2048
_BETA = 0.25


def _vq_body(sx_ref, cb2_ref, xf_ref, cb_ref,
             q_ref, loss_ref, perp_ref,
             counts_ref, acc_ref):
    step = pl.program_id(0)
    nsteps = pl.num_programs(0)

    @pl.when(step == 0)
    def _init():
        counts_ref[...] = jnp.zeros_like(counts_ref)
        acc_ref[0] = 0.0

    xb = xf_ref[...]
    cb = cb_ref[...]
    sx = sx_ref[...]
    cb2 = cb2_ref[...]

    mm = jax.lax.dot_general(
        xb, cb, dimension_numbers=(((1,), (1,)), ((), ())),
        preferred_element_type=jnp.float32)
    dist = (sx + cb2) - 2.0 * mm

    mn = jnp.min(dist, axis=1, keepdims=True)
    iota = jax.lax.broadcasted_iota(jnp.int32, (_TB, _CB), 1)
    idxv = jnp.min(jnp.where(dist == mn, iota, _CB), axis=1, keepdims=True)
    onehot = (iota == idxv).astype(jnp.float32)

    q = jax.lax.dot_general(
        onehot, cb, dimension_numbers=(((1,), (0,)), ((), ())),
        preferred_element_type=jnp.float32)
    q_ref[...] = q

    acc_ref[0] += jnp.sum((q - xb) ** 2)
    counts_ref[...] += jnp.sum(onehot, axis=0, keepdims=True)

    @pl.when(step == nsteps - 1)
    def _fin():
        p = counts_ref[...] * (1.0 / _N)
        ent = jnp.sum(p * jnp.log(p + 1e-10))
        perp_ref[0, 0] = jnp.exp(-ent)
        m = acc_ref[0] * (1.0 / (_N * _D))
        loss_ref[0, 0] = m + _BETA * m


@functools.partial(jax.jit, static_argnames=("interpret",))
def _vq_call(flat, sx, cb2, codebook, interpret=False):
    nsteps = _N // _TB
    q, loss, perp = pl.pallas_call(
        _vq_body,
        grid=(nsteps,),
        in_specs=[
            pl.BlockSpec((_TB, 1), lambda i: (i, 0)),
            pl.BlockSpec((1, _CB), lambda i: (0, 0)),
            pl.BlockSpec((_TB, _D), lambda i: (i, 0)),
            pl.BlockSpec((_CB, _D), lambda i: (0, 0)),
        ],
        out_specs=[
            pl.BlockSpec((_TB, _D), lambda i: (i, 0)),
            pl.BlockSpec(memory_space=pltpu.SMEM),
            pl.BlockSpec(memory_space=pltpu.SMEM),
        ],
        out_shape=[
            jax.ShapeDtypeStruct((_N, _D), jnp.float32),
            jax.ShapeDtypeStruct((1, 1), jnp.float32),
            jax.ShapeDtypeStruct((1, 1), jnp.float32),
        ],
        scratch_shapes=[
            pltpu.VMEM((1, _CB), jnp.float32),
            pltpu.SMEM((1,), jnp.float32),
        ],
        interpret=interpret,
    )(sx, cb2, flat, codebook)
    return q, loss, perp


def kernel(x, codebook):
    xt = jnp.transpose(x, (0, 2, 1))
    flat = xt.reshape(-1, _D)
    sx = jnp.sum(flat ** 2, axis=1, keepdims=True)
    cb2 = jnp.sum(codebook ** 2, axis=1)[None, :]
    q, loss, perp = _vq_call(flat, sx, cb2, codebook)
    content = jnp.transpose(q.reshape(16, 2048, _D), (0, 2, 1))
    return content, loss.reshape(()), perp.reshape(())
